# baseline (speedup 1.0000x reference)
import jax
import jax.numpy as jnp
from jax.experimental import pallas as pl
from jax.experimental.pallas import tpu as pltpu

N_ITEMS = 16384
EMB_DIM = 64
KNN_K = 10
K_PAD = 16
BLOCK_R = 128
NORM_BLOCK = 1024


def _normalize_body(x_ref, xn_ref):
    x = x_ref[...]
    nrm = jnp.sqrt(jnp.sum(x * x, axis=1, keepdims=True))
    xn_ref[...] = x / nrm


SEGS = 128
SEG_W = 128
SEG_T = 4


def _extract_topk(sim, idx_src, big, mask_val=-3.0):
    vals, idxs = [], []
    for _ in range(KNN_K):
        m = jnp.max(sim, axis=1, keepdims=True)
        eq = sim == m
        idx = jnp.min(jnp.where(eq, idx_src, big), axis=1, keepdims=True)
        vals.append(m)
        idxs.append(idx)
        sim = jnp.where(idx_src == idx, mask_val, sim)
    return jnp.concatenate(vals, axis=1), jnp.concatenate(idxs, axis=1)


def _topk_body(xb_ref, xt_ref, val_ref, idx_ref, lap_ref):
    xb = xb_ref[...]
    xt = xt_ref[...]
    sim = jnp.dot(xb, xt, preferred_element_type=jnp.float32)

    sim3 = sim.reshape(BLOCK_R, SEGS, SEG_W)
    lane = jax.lax.broadcasted_iota(jnp.int32, (BLOCK_R, SEGS, SEG_W), 2)
    segbase = jax.lax.broadcasted_iota(jnp.int32, (BLOCK_R, SEGS), 1) * SEG_W
    cvals, cidxs = [], []
    for _ in range(SEG_T):
        m = jnp.max(sim3, axis=2, keepdims=True)
        eq = sim3 == m
        iw = jnp.min(jnp.where(eq, lane, SEG_W), axis=2, keepdims=True)
        cvals.append(m[..., 0])
        cidxs.append(segbase + iw[..., 0])
        sim3 = jnp.where(lane == iw, -3.0, sim3)
    cand_v = jnp.concatenate(cvals, axis=1)
    cand_i = jnp.concatenate(cidxs, axis=1)

    vals, idxs = _extract_topk(cand_v, cand_i, N_ITEMS)

    pad_v = jnp.zeros((BLOCK_R, K_PAD - KNN_K), dtype=jnp.float32)
    pad_i = jnp.zeros((BLOCK_R, K_PAD - KNN_K), dtype=jnp.int32)
    val_ref[...] = jnp.concatenate([vals, pad_v], axis=1)
    idx_ref[...] = jnp.concatenate([idxs, pad_i], axis=1)
    rs = jnp.full((BLOCK_R, K_PAD), 10.0 + 1e-07, dtype=jnp.float32)
    ris = jnp.power(rs, -0.5)
    lap_ref[...] = ris * ris

    v10 = vals[:, KNN_K - 1:KNN_K]
    flag = jnp.any(cvals[SEG_T - 1] >= v10)

    @pl.when(flag)
    def _fallback():
        simf = jnp.dot(xb, xt, preferred_element_type=jnp.float32)
        col = jax.lax.broadcasted_iota(jnp.int32, (BLOCK_R, N_ITEMS), 1)
        fv, fi = _extract_topk(simf, col, N_ITEMS)
        val_ref[...] = jnp.concatenate([fv, pad_v], axis=1)
        idx_ref[...] = jnp.concatenate([fi, pad_i], axis=1)


def kernel(mm_embeddings):
    n = N_ITEMS
    xn = pl.pallas_call(
        _normalize_body,
        grid=(n // NORM_BLOCK,),
        in_specs=[pl.BlockSpec((NORM_BLOCK, EMB_DIM), lambda i: (i, 0))],
        out_specs=pl.BlockSpec((NORM_BLOCK, EMB_DIM), lambda i: (i, 0)),
        out_shape=jax.ShapeDtypeStruct((n, EMB_DIM), jnp.float32),
    )(mm_embeddings)
    xt = xn.T

    vals, idxs, lap = pl.pallas_call(
        _topk_body,
        grid=(n // BLOCK_R,),
        in_specs=[
            pl.BlockSpec((BLOCK_R, EMB_DIM), lambda i: (i, 0)),
            pl.BlockSpec((EMB_DIM, n), lambda i: (0, 0)),
        ],
        out_specs=[
            pl.BlockSpec((BLOCK_R, K_PAD), lambda i: (i, 0)),
            pl.BlockSpec((BLOCK_R, K_PAD), lambda i: (i, 0)),
            pl.BlockSpec((BLOCK_R, K_PAD), lambda i: (i, 0)),
        ],
        out_shape=[
            jax.ShapeDtypeStruct((n, K_PAD), jnp.float32),
            jax.ShapeDtypeStruct((n, K_PAD), jnp.int32),
            jax.ShapeDtypeStruct((n, K_PAD), jnp.float32),
        ],
        compiler_params=pltpu.CompilerParams(
            dimension_semantics=("parallel",),
        ),
    )(xn, xt)

    knn_val = vals[:, :KNN_K]
    cols = idxs[:, :KNN_K].reshape(-1)
    rows = jnp.repeat(jnp.arange(n, dtype=jnp.int32), KNN_K)
    indices = jnp.stack((rows, cols), axis=0)
    values = lap[:, :KNN_K].reshape(-1)
    return knn_val, indices, values


# speedup vs baseline: 3.2122x; 3.2122x over previous
import jax
import jax.numpy as jnp
from jax.experimental import pallas as pl
from jax.experimental.pallas import tpu as pltpu

N_ITEMS = 16384
EMB_DIM = 64
KNN_K = 10
K_PAD = 16
BLOCK_R = 128
NORM_BLOCK = 1024


def _normalize_body(x_ref, xn_ref):
    x = x_ref[...]
    nrm = jnp.sqrt(jnp.sum(x * x, axis=1, keepdims=True))
    xn_ref[...] = x / nrm


SEGS = 128
SEG_W = 128
SEG_T = 4


def _extract_topk(sim, idx_src, big, mask_val=-3.0):
    vals, idxs = [], []
    for _ in range(KNN_K):
        m = jnp.max(sim, axis=1, keepdims=True)
        eq = sim == m
        idx = jnp.min(jnp.where(eq, idx_src, big), axis=1, keepdims=True)
        vals.append(m)
        idxs.append(idx)
        sim = jnp.where(idx_src == idx, mask_val, sim)
    return jnp.concatenate(vals, axis=1), jnp.concatenate(idxs, axis=1)


def _topk_body(xb_ref, xt_ref, val_ref, idx_ref, lap_ref):
    xb = xb_ref[...]
    xt = xt_ref[...]
    sim = jnp.dot(xb, xt, preferred_element_type=jnp.float32)

    sim3 = sim.reshape(BLOCK_R, SEG_W, SEGS)
    wpos = jax.lax.broadcasted_iota(jnp.int32, (BLOCK_R, SEG_W, SEGS), 1)
    soff = jax.lax.broadcasted_iota(jnp.int32, (BLOCK_R, SEGS), 1)
    cvals, cidxs = [], []
    for _ in range(SEG_T):
        m = jnp.max(sim3, axis=1, keepdims=True)
        eq = sim3 == m
        iw = jnp.min(jnp.where(eq, wpos, SEG_W), axis=1, keepdims=True)
        cvals.append(m[:, 0, :])
        cidxs.append(iw[:, 0, :] * SEGS + soff)
        sim3 = jnp.where(wpos == iw, -3.0, sim3)
    cand_v = jnp.concatenate(cvals, axis=1)
    cand_i = jnp.concatenate(cidxs, axis=1)

    vals, idxs = _extract_topk(cand_v, cand_i, N_ITEMS)

    pad_v = jnp.zeros((BLOCK_R, K_PAD - KNN_K), dtype=jnp.float32)
    pad_i = jnp.zeros((BLOCK_R, K_PAD - KNN_K), dtype=jnp.int32)
    val_ref[...] = jnp.concatenate([vals, pad_v], axis=1)
    idx_ref[...] = jnp.concatenate([idxs, pad_i], axis=1)
    rs = jnp.full((BLOCK_R, K_PAD), 10.0 + 1e-07, dtype=jnp.float32)
    ris = jnp.power(rs, -0.5)
    lap_ref[...] = ris * ris

    v10 = vals[:, KNN_K - 1:KNN_K]
    flag = jnp.any(cvals[SEG_T - 1] >= v10)

    @pl.when(flag)
    def _fallback():
        simf = jnp.dot(xb, xt, preferred_element_type=jnp.float32)
        col = jax.lax.broadcasted_iota(jnp.int32, (BLOCK_R, N_ITEMS), 1)
        fv, fi = _extract_topk(simf, col, N_ITEMS)
        val_ref[...] = jnp.concatenate([fv, pad_v], axis=1)
        idx_ref[...] = jnp.concatenate([fi, pad_i], axis=1)


def kernel(mm_embeddings):
    n = N_ITEMS
    xn = pl.pallas_call(
        _normalize_body,
        grid=(n // NORM_BLOCK,),
        in_specs=[pl.BlockSpec((NORM_BLOCK, EMB_DIM), lambda i: (i, 0))],
        out_specs=pl.BlockSpec((NORM_BLOCK, EMB_DIM), lambda i: (i, 0)),
        out_shape=jax.ShapeDtypeStruct((n, EMB_DIM), jnp.float32),
    )(mm_embeddings)
    xt = xn.T

    vals, idxs, lap = pl.pallas_call(
        _topk_body,
        grid=(n // BLOCK_R,),
        in_specs=[
            pl.BlockSpec((BLOCK_R, EMB_DIM), lambda i: (i, 0)),
            pl.BlockSpec((EMB_DIM, n), lambda i: (0, 0)),
        ],
        out_specs=[
            pl.BlockSpec((BLOCK_R, K_PAD), lambda i: (i, 0)),
            pl.BlockSpec((BLOCK_R, K_PAD), lambda i: (i, 0)),
            pl.BlockSpec((BLOCK_R, K_PAD), lambda i: (i, 0)),
        ],
        out_shape=[
            jax.ShapeDtypeStruct((n, K_PAD), jnp.float32),
            jax.ShapeDtypeStruct((n, K_PAD), jnp.int32),
            jax.ShapeDtypeStruct((n, K_PAD), jnp.float32),
        ],
        compiler_params=pltpu.CompilerParams(
            dimension_semantics=("parallel",),
        ),
    )(xn, xt)

    knn_val = vals[:, :KNN_K]
    cols = idxs[:, :KNN_K].reshape(-1)
    rows = jnp.repeat(jnp.arange(n, dtype=jnp.int32), KNN_K)
    indices = jnp.stack((rows, cols), axis=0)
    values = lap[:, :KNN_K].reshape(-1)
    return knn_val, indices, values
